# baseline (speedup 1.0000x reference)
import functools

import jax
import jax.numpy as jnp
from jax import lax
from jax.experimental import pallas as pl
from jax.experimental.pallas import tpu as pltpu
from jax.experimental.pallas import tpu_sc as plsc

E = 64
TOPK = 8
D = 2048
FF = 5632
T = 4096

M = 256
S = T * TOPK
NT = (S + E * (M - 1) + M - 1) // M
SPAD = NT * M
F_BLK = 1408
NF = FF // F_BLK
DH_BLK = 1024
NDH = D // DH_BLK
TT1 = 512
C2 = 512
F7 = 256
TT7 = 1024
TTC = 256

NEG = -1e30

SC_NC = 2
SC_NS = 16
NW = SC_NC * SC_NS
PER_W = S // NW
B_SC = 16
N_IT = PER_W // B_SC


def _router_body(x_ref, rw_ref, idx_ref, w_ref):
    logits = lax.dot_general(x_ref[:], rw_ref[:], (((1,), (1,)), ((), ())),
                             preferred_element_type=jnp.float32)
    iota_e = lax.broadcasted_iota(jnp.int32, (TT1, E), 1)
    iota_k = lax.broadcasted_iota(jnp.int32, (TT1, TOPK), 1)
    vals = jnp.zeros((TT1, TOPK), jnp.float32)
    idxs = jnp.zeros((TT1, TOPK), jnp.int32)
    l = logits
    for k in range(TOPK):
        m = jnp.max(l, axis=1, keepdims=True)
        amax = jnp.min(jnp.where(l == m, iota_e, E), axis=1, keepdims=True)
        vals = jnp.where(iota_k == k, m, vals)
        idxs = jnp.where(iota_k == k, amax, idxs)
        l = jnp.where(iota_e == amax, NEG, l)
    ex = jnp.exp(vals - vals[:, 0:1])
    w_ref[:] = ex / jnp.sum(ex, axis=1, keepdims=True)
    idx_ref[:] = idxs


def _router(x, router_weight):
    return pl.pallas_call(
        _router_body,
        grid=(T // TT1,),
        in_specs=[
            pl.BlockSpec((TT1, D), lambda i: (i, 0)),
            pl.BlockSpec((E, D), lambda i: (0, 0)),
        ],
        out_specs=[
            pl.BlockSpec((TT1, TOPK), lambda i: (i, 0)),
            pl.BlockSpec((TT1, TOPK), lambda i: (i, 0)),
        ],
        out_shape=[
            jax.ShapeDtypeStruct((T, TOPK), jnp.int32),
            jax.ShapeDtypeStruct((T, TOPK), jnp.float32),
        ],
    )(x, router_weight)


def _meta_body(idx_ref, pos_ref, te_ref, valid_ref):
    idx = idx_ref[:]
    eqf = (idx[:, :, None] ==
           lax.broadcasted_iota(jnp.int32, (T, TOPK, E), 2)).astype(jnp.float32)
    a = jnp.sum(eqf, axis=1)
    ltri = (lax.broadcasted_iota(jnp.int32, (C2, C2), 0) >
            lax.broadcasted_iota(jnp.int32, (C2, C2), 1)).astype(jnp.float32)
    carry = jnp.zeros((1, E), jnp.float32)
    chunks = []
    for c in range(T // C2):
        ac = lax.slice(a, (c * C2, 0), ((c + 1) * C2, E))
        chunks.append(lax.dot_general(ltri, ac, (((1,), (0,)), ((), ())),
                                      preferred_element_type=jnp.float32)
                      + carry)
        carry = carry + jnp.sum(ac, axis=0, keepdims=True)
    rank = jnp.concatenate(chunks, axis=0)
    counts = carry
    counts_pad = jnp.floor((counts + (M - 1)) * (1.0 / M)) * M
    sut = (lax.broadcasted_iota(jnp.int32, (E, E), 0) <
           lax.broadcasted_iota(jnp.int32, (E, E), 1)).astype(jnp.float32)
    offs = lax.dot_general(counts_pad, sut, (((1,), (0,)), ((), ())),
                           preferred_element_type=jnp.float32)
    ends = offs + counts_pad
    total = jnp.sum(counts_pad, axis=1, keepdims=True)
    rank_sel = jnp.sum(eqf * rank[:, None, :], axis=2)
    offs_sel = jnp.sum(eqf * offs[:, None, :], axis=2)
    pos_ref[:] = (rank_sel + offs_sel).astype(jnp.int32)
    ident = (lax.broadcasted_iota(jnp.int32, (E, E), 0) ==
             lax.broadcasted_iota(jnp.int32, (E, E), 1)).astype(jnp.float32)
    ends_t = lax.dot_general(ident, ends, (((1,), (1,)), ((), ())),
                             preferred_element_type=jnp.float32)
    jstart = (lax.broadcasted_iota(jnp.int32, (1, NT), 1).astype(jnp.float32)
              * float(M))
    te_f = jnp.sum((ends_t <= jstart).astype(jnp.float32), axis=0,
                   keepdims=True)
    te_ref[:] = jnp.minimum(te_f, E - 1).astype(jnp.int32)
    valid_ref[:] = (jstart < total).astype(jnp.int32)


def _metadata(topk_idx):
    return pl.pallas_call(
        _meta_body,
        out_shape=[
            jax.ShapeDtypeStruct((T, TOPK), jnp.int32),
            jax.ShapeDtypeStruct((1, NT), jnp.int32),
            jax.ShapeDtypeStruct((1, NT), jnp.int32),
        ],
    )(topk_idx)


def _sc_scatter_body(x_hbm, pos_hbm, out_hbm, pos_v, buf, sem_g, sem_s):
    wid = lax.axis_index("s") * SC_NC + lax.axis_index("c")
    base = wid * PER_W
    pltpu.sync_copy(pos_hbm.at[pl.ds(base, PER_W)], pos_v)
    lane = lax.iota(jnp.int32, B_SC)

    def body(i, _):
        sv = base + i * B_SC + lane
        tv = lax.shift_right_logical(sv, 3)
        pltpu.async_copy(x_hbm.at[tv], buf, sem_g).wait()
        posv = pos_v[pl.ds(i * B_SC, B_SC)]
        pltpu.async_copy(buf, out_hbm.at[posv], sem_s).wait()
        return 0

    lax.fori_loop(0, N_IT, body, 0)


D2 = D // 2


def _bf16_rows_to_i32(a):
    return lax.bitcast_convert_type(
        a.reshape(a.shape[0], a.shape[1] // 2, 2), jnp.int32)


def _i32_rows_to_bf16(a):
    return lax.bitcast_convert_type(a, jnp.bfloat16).reshape(a.shape[0], -1)


def _sc_scatter(x_i32, pos_flat):
    return pl.kernel(
        _sc_scatter_body,
        out_type=jax.ShapeDtypeStruct((SPAD, D2), jnp.int32),
        mesh=plsc.VectorSubcoreMesh(core_axis_name="c", subcore_axis_name="s"),
        scratch_types=[
            pltpu.VMEM((PER_W,), jnp.int32),
            pltpu.VMEM((B_SC, D2), jnp.int32),
            pltpu.SemaphoreType.DMA,
            pltpu.SemaphoreType.DMA,
        ],
    )(x_i32, pos_flat)


def _sc_gather_body(ys_hbm, pos_hbm, out_hbm, pos_v, buf, sem_g, sem_s):
    wid = lax.axis_index("s") * SC_NC + lax.axis_index("c")
    base = wid * PER_W
    pltpu.sync_copy(pos_hbm.at[pl.ds(base, PER_W)], pos_v)

    def body(i, _):
        posv = pos_v[pl.ds(i * B_SC, B_SC)]
        pltpu.async_copy(ys_hbm.at[posv], buf, sem_g).wait()
        pltpu.async_copy(buf, out_hbm.at[pl.ds(base + i * B_SC, B_SC)],
                         sem_s).wait()
        return 0

    lax.fori_loop(0, N_IT, body, 0)


def _sc_gather(y_sorted_i32, pos_flat):
    return pl.kernel(
        _sc_gather_body,
        out_type=jax.ShapeDtypeStruct((S, D2), jnp.int32),
        mesh=plsc.VectorSubcoreMesh(core_axis_name="c", subcore_axis_name="s"),
        scratch_types=[
            pltpu.VMEM((PER_W,), jnp.int32),
            pltpu.VMEM((B_SC, D2), jnp.int32),
            pltpu.SemaphoreType.DMA,
            pltpu.SemaphoreType.DMA,
        ],
    )(y_sorted_i32, pos_flat)


def _gateup_body(te_sm, valid_sm, gs_sm, us_sm, xs_ref, gw_ref, uw_ref, h_ref):
    m = pl.program_id(1)

    @pl.when(valid_sm[m] == 1)
    def _():
        e = te_sm[m]
        xb = xs_ref[:]
        gw = gw_ref[0].astype(jnp.bfloat16)
        uw = uw_ref[0].astype(jnp.bfloat16)
        g = lax.dot_general(xb, gw, (((1,), (1,)), ((), ())),
                            preferred_element_type=jnp.float32) * gs_sm[e]
        u = lax.dot_general(xb, uw, (((1,), (1,)), ((), ())),
                            preferred_element_type=jnp.float32) * us_sm[e]
        act = g * (1.0 / (1.0 + jnp.exp(-g)))
        h_ref[:] = (act * u).astype(jnp.bfloat16)


def _gateup(tile_ex, valid, gate_s, up_s, xs, gate_w, up_w):
    grid_spec = pltpu.PrefetchScalarGridSpec(
        num_scalar_prefetch=4,
        grid=(NF, NT),
        in_specs=[
            pl.BlockSpec((M, D), lambda f, m, te, va, gs, us: (m, 0)),
            pl.BlockSpec((1, F_BLK, D),
                         lambda f, m, te, va, gs, us: (te[m], f, 0)),
            pl.BlockSpec((1, F_BLK, D),
                         lambda f, m, te, va, gs, us: (te[m], f, 0)),
        ],
        out_specs=pl.BlockSpec((M, F_BLK), lambda f, m, te, va, gs, us: (m, f)),
    )
    return pl.pallas_call(
        _gateup_body,
        grid_spec=grid_spec,
        out_shape=jax.ShapeDtypeStruct((SPAD, FF), jnp.bfloat16),
    )(tile_ex, valid, gate_s, up_s, xs, gate_w, up_w)


def _down_body(te_sm, valid_sm, ds_sm, h_ref, dw_ref, y_ref):
    m = pl.program_id(1)

    @pl.when(valid_sm[m] == 1)
    def _():
        e = te_sm[m]
        h = h_ref[:]
        dw = dw_ref[0].astype(jnp.bfloat16)
        y = lax.dot_general(h, dw, (((1,), (1,)), ((), ())),
                            preferred_element_type=jnp.float32) * ds_sm[e]
        y_ref[:] = y.astype(jnp.bfloat16)


def _down(tile_ex, valid, down_s, h, down_w):
    grid_spec = pltpu.PrefetchScalarGridSpec(
        num_scalar_prefetch=3,
        grid=(NDH, NT),
        in_specs=[
            pl.BlockSpec((M, FF), lambda dh, m, te, va, ds: (m, 0)),
            pl.BlockSpec((1, DH_BLK, FF),
                         lambda dh, m, te, va, ds: (te[m], dh, 0)),
        ],
        out_specs=pl.BlockSpec((M, DH_BLK), lambda dh, m, te, va, ds: (m, dh)),
    )
    return pl.pallas_call(
        _down_body,
        grid_spec=grid_spec,
        out_shape=jax.ShapeDtypeStruct((SPAD, D), jnp.bfloat16),
    )(tile_ex, valid, down_s, h, down_w)


def _shared_body(x_ref, sg_ref, su_ref, sd_ref, out_ref):
    f = pl.program_id(1)

    @pl.when(f == 0)
    def _():
        out_ref[:] = jnp.zeros_like(out_ref)

    xb = x_ref[:]
    sg = sg_ref[:].astype(jnp.bfloat16)
    su = su_ref[:].astype(jnp.bfloat16)
    sd = sd_ref[:].astype(jnp.bfloat16)
    g = lax.dot_general(xb, sg, (((1,), (1,)), ((), ())),
                        preferred_element_type=jnp.float32)
    u = lax.dot_general(xb, su, (((1,), (1,)), ((), ())),
                        preferred_element_type=jnp.float32)
    h = ((g * (1.0 / (1.0 + jnp.exp(-g)))) * u).astype(jnp.bfloat16)
    out_ref[:] += lax.dot_general(h, sd, (((1,), (1,)), ((), ())),
                                  preferred_element_type=jnp.float32)


def _shared(x, shared_gate, shared_up, shared_down):
    return pl.pallas_call(
        _shared_body,
        grid=(T // TT7, FF // F7),
        in_specs=[
            pl.BlockSpec((TT7, D), lambda t, f: (t, 0)),
            pl.BlockSpec((F7, D), lambda t, f: (f, 0)),
            pl.BlockSpec((F7, D), lambda t, f: (f, 0)),
            pl.BlockSpec((D, F7), lambda t, f: (0, f)),
        ],
        out_specs=pl.BlockSpec((TT7, D), lambda t, f: (t, 0)),
        out_shape=jax.ShapeDtypeStruct((T, D), jnp.float32),
    )(x, shared_gate, shared_up, shared_down)


def _combine_body(sh_ref, y_ref, w_ref, out_ref):
    acc = sh_ref[:]
    for k in range(TOPK):
        acc = acc + w_ref[:, k:k + 1] * y_ref[:, k, :].astype(jnp.float32)
    out_ref[:] = acc


def _combine(shared_out, y3, w):
    return pl.pallas_call(
        _combine_body,
        grid=(T // TTC,),
        in_specs=[
            pl.BlockSpec((TTC, D), lambda i: (i, 0)),
            pl.BlockSpec((TTC, TOPK, D), lambda i: (i, 0, 0)),
            pl.BlockSpec((TTC, TOPK), lambda i: (i, 0)),
        ],
        out_specs=pl.BlockSpec((TTC, D), lambda i: (i, 0)),
        out_shape=jax.ShapeDtypeStruct((T, D), jnp.float32),
    )(shared_out, y3, w)


def kernel(x, router_weight, gate_w, up_w, down_w, gate_s, up_s, down_s,
           shared_gate, shared_up, shared_down):
    topk_idx, routing_w = _router(x, router_weight)
    pos, tile_ex, valid = _metadata(topk_idx)
    pos_flat = pos.reshape(S)
    tile_ex = tile_ex.reshape(NT)
    valid = valid.reshape(NT)

    xb = x.astype(jnp.bfloat16)
    xs = _i32_rows_to_bf16(_sc_scatter(_bf16_rows_to_i32(xb), pos_flat))
    h = _gateup(tile_ex, valid, gate_s, up_s, xs, gate_w, up_w)
    ys = _down(tile_ex, valid, down_s, h, down_w)
    yg = _i32_rows_to_bf16(_sc_gather(_bf16_rows_to_i32(ys), pos_flat))

    sh = _shared(xb, shared_gate, shared_up, shared_down)
    return _combine(sh, yg.reshape(T, TOPK, D), routing_w)


# speedup vs baseline: 2.3098x; 2.3098x over previous
import functools

import jax
import jax.numpy as jnp
from jax import lax
from jax.experimental import pallas as pl
from jax.experimental.pallas import tpu as pltpu
from jax.experimental.pallas import tpu_sc as plsc

E = 64
TOPK = 8
D = 2048
FF = 5632
T = 4096

M = 256
S = T * TOPK
NT = (S + E * (M - 1) + M - 1) // M
SPAD = NT * M
F_BLK = 1408
NF = FF // F_BLK
DH_BLK = 1024
NDH = D // DH_BLK
TT1 = 512
C2 = 512
F7 = 256
TT7 = 1024
TTC = 256

NEG = -1e30

SC_NC = 2
SC_NS = 16
NW = SC_NC * SC_NS
PER_W = S // NW
B_SC = 16
N_IT = PER_W // B_SC


def _router_body(x_ref, rw_ref, idx_ref, w_ref):
    logits = lax.dot_general(x_ref[:], rw_ref[:], (((1,), (1,)), ((), ())),
                             preferred_element_type=jnp.float32)
    iota_e = lax.broadcasted_iota(jnp.int32, (TT1, E), 1)
    iota_k = lax.broadcasted_iota(jnp.int32, (TT1, TOPK), 1)
    vals = jnp.zeros((TT1, TOPK), jnp.float32)
    idxs = jnp.zeros((TT1, TOPK), jnp.int32)
    l = logits
    for k in range(TOPK):
        m = jnp.max(l, axis=1, keepdims=True)
        amax = jnp.min(jnp.where(l == m, iota_e, E), axis=1, keepdims=True)
        vals = jnp.where(iota_k == k, m, vals)
        idxs = jnp.where(iota_k == k, amax, idxs)
        l = jnp.where(iota_e == amax, NEG, l)
    ex = jnp.exp(vals - vals[:, 0:1])
    w_ref[:] = ex / jnp.sum(ex, axis=1, keepdims=True)
    idx_ref[:] = idxs


def _router(x, router_weight):
    return pl.pallas_call(
        _router_body,
        grid=(T // TT1,),
        in_specs=[
            pl.BlockSpec((TT1, D), lambda i: (i, 0)),
            pl.BlockSpec((E, D), lambda i: (0, 0)),
        ],
        out_specs=[
            pl.BlockSpec((TT1, TOPK), lambda i: (i, 0)),
            pl.BlockSpec((TT1, TOPK), lambda i: (i, 0)),
        ],
        out_shape=[
            jax.ShapeDtypeStruct((T, TOPK), jnp.int32),
            jax.ShapeDtypeStruct((T, TOPK), jnp.float32),
        ],
    )(x, router_weight)


def _meta_body(idx_ref, pos_ref, te_ref, valid_ref):
    idx = idx_ref[:]
    eqf = (idx[:, :, None] ==
           lax.broadcasted_iota(jnp.int32, (T, TOPK, E), 2)).astype(jnp.float32)
    a = jnp.sum(eqf, axis=1)
    ltri = (lax.broadcasted_iota(jnp.int32, (C2, C2), 0) >
            lax.broadcasted_iota(jnp.int32, (C2, C2), 1)).astype(jnp.float32)
    carry = jnp.zeros((1, E), jnp.float32)
    chunks = []
    for c in range(T // C2):
        ac = lax.slice(a, (c * C2, 0), ((c + 1) * C2, E))
        chunks.append(lax.dot_general(ltri, ac, (((1,), (0,)), ((), ())),
                                      preferred_element_type=jnp.float32)
                      + carry)
        carry = carry + jnp.sum(ac, axis=0, keepdims=True)
    rank = jnp.concatenate(chunks, axis=0)
    counts = carry
    counts_pad = jnp.floor((counts + (M - 1)) * (1.0 / M)) * M
    sut = (lax.broadcasted_iota(jnp.int32, (E, E), 0) <
           lax.broadcasted_iota(jnp.int32, (E, E), 1)).astype(jnp.float32)
    offs = lax.dot_general(counts_pad, sut, (((1,), (0,)), ((), ())),
                           preferred_element_type=jnp.float32)
    ends = offs + counts_pad
    total = jnp.sum(counts_pad, axis=1, keepdims=True)
    rank_sel = jnp.sum(eqf * rank[:, None, :], axis=2)
    offs_sel = jnp.sum(eqf * offs[:, None, :], axis=2)
    pos_ref[:] = (rank_sel + offs_sel).astype(jnp.int32)
    ident = (lax.broadcasted_iota(jnp.int32, (E, E), 0) ==
             lax.broadcasted_iota(jnp.int32, (E, E), 1)).astype(jnp.float32)
    ends_t = lax.dot_general(ident, ends, (((1,), (1,)), ((), ())),
                             preferred_element_type=jnp.float32)
    jstart = (lax.broadcasted_iota(jnp.int32, (1, NT), 1).astype(jnp.float32)
              * float(M))
    te_f = jnp.sum((ends_t <= jstart).astype(jnp.float32), axis=0,
                   keepdims=True)
    te_ref[:] = jnp.minimum(te_f, E - 1).astype(jnp.int32)
    valid_ref[:] = (jstart < total).astype(jnp.int32)


def _metadata(topk_idx):
    return pl.pallas_call(
        _meta_body,
        out_shape=[
            jax.ShapeDtypeStruct((T, TOPK), jnp.int32),
            jax.ShapeDtypeStruct((1, NT), jnp.int32),
            jax.ShapeDtypeStruct((1, NT), jnp.int32),
        ],
    )(topk_idx)


def _sc_scatter_body(x_hbm, pos_hbm, out_hbm, pos_v, buf, sem_g, sem_s):
    wid = lax.axis_index("s") * SC_NC + lax.axis_index("c")
    base = wid * PER_W
    pltpu.sync_copy(pos_hbm.at[pl.ds(base, PER_W)], pos_v)
    lane = lax.iota(jnp.int32, B_SC)

    def body(i, _):
        sv = base + i * B_SC + lane
        tv = lax.shift_right_logical(sv, 3)
        pltpu.async_copy(x_hbm.at[tv], buf, sem_g).wait()
        posv = pos_v[pl.ds(i * B_SC, B_SC)]
        pltpu.async_copy(buf, out_hbm.at[posv], sem_s).wait()
        return 0

    lax.fori_loop(0, N_IT, body, 0)


def _sc_scatter(x, pos_flat):
    return pl.kernel(
        _sc_scatter_body,
        out_type=jax.ShapeDtypeStruct((SPAD, D), jnp.float32),
        mesh=plsc.VectorSubcoreMesh(core_axis_name="c", subcore_axis_name="s"),
        scratch_types=[
            pltpu.VMEM((PER_W,), jnp.int32),
            pltpu.VMEM((B_SC, D), jnp.float32),
            pltpu.SemaphoreType.DMA,
            pltpu.SemaphoreType.DMA,
        ],
    )(x, pos_flat)


def _sc_gather_body(ys_hbm, pos_hbm, out_hbm, pos_v, buf, sem_g, sem_s):
    wid = lax.axis_index("s") * SC_NC + lax.axis_index("c")
    base = wid * PER_W
    pltpu.sync_copy(pos_hbm.at[pl.ds(base, PER_W)], pos_v)

    def body(i, _):
        posv = pos_v[pl.ds(i * B_SC, B_SC)]
        pltpu.async_copy(ys_hbm.at[posv], buf, sem_g).wait()
        pltpu.async_copy(buf, out_hbm.at[pl.ds(base + i * B_SC, B_SC)],
                         sem_s).wait()
        return 0

    lax.fori_loop(0, N_IT, body, 0)


def _sc_gather(y_sorted, pos_flat):
    return pl.kernel(
        _sc_gather_body,
        out_type=jax.ShapeDtypeStruct((S, D), jnp.float32),
        mesh=plsc.VectorSubcoreMesh(core_axis_name="c", subcore_axis_name="s"),
        scratch_types=[
            pltpu.VMEM((PER_W,), jnp.int32),
            pltpu.VMEM((B_SC, D), jnp.float32),
            pltpu.SemaphoreType.DMA,
            pltpu.SemaphoreType.DMA,
        ],
    )(y_sorted, pos_flat)


def _gateup_body(te_sm, valid_sm, gs_sm, us_sm, xs_ref, gw_ref, uw_ref, h_ref):
    m = pl.program_id(1)

    @pl.when(valid_sm[m] == 1)
    def _():
        e = te_sm[m]
        xb = xs_ref[:].astype(jnp.bfloat16)
        gw = gw_ref[0].astype(jnp.bfloat16)
        uw = uw_ref[0].astype(jnp.bfloat16)
        g = lax.dot_general(xb, gw, (((1,), (1,)), ((), ())),
                            preferred_element_type=jnp.float32) * gs_sm[e]
        u = lax.dot_general(xb, uw, (((1,), (1,)), ((), ())),
                            preferred_element_type=jnp.float32) * us_sm[e]
        act = g * (1.0 / (1.0 + jnp.exp(-g)))
        h_ref[:] = (act * u).astype(jnp.bfloat16)


def _gateup(tile_ex, valid, gate_s, up_s, xs, gate_w, up_w):
    grid_spec = pltpu.PrefetchScalarGridSpec(
        num_scalar_prefetch=4,
        grid=(NF, NT),
        in_specs=[
            pl.BlockSpec((M, D), lambda f, m, te, va, gs, us: (m, 0)),
            pl.BlockSpec((1, F_BLK, D),
                         lambda f, m, te, va, gs, us: (te[m], f, 0)),
            pl.BlockSpec((1, F_BLK, D),
                         lambda f, m, te, va, gs, us: (te[m], f, 0)),
        ],
        out_specs=pl.BlockSpec((M, F_BLK), lambda f, m, te, va, gs, us: (m, f)),
    )
    return pl.pallas_call(
        _gateup_body,
        grid_spec=grid_spec,
        out_shape=jax.ShapeDtypeStruct((SPAD, FF), jnp.bfloat16),
    )(tile_ex, valid, gate_s, up_s, xs, gate_w, up_w)


def _down_body(te_sm, valid_sm, ds_sm, h_ref, dw_ref, y_ref):
    m = pl.program_id(1)

    @pl.when(valid_sm[m] == 1)
    def _():
        e = te_sm[m]
        h = h_ref[:]
        dw = dw_ref[0].astype(jnp.bfloat16)
        y_ref[:] = lax.dot_general(h, dw, (((1,), (1,)), ((), ())),
                                   preferred_element_type=jnp.float32) * ds_sm[e]


def _down(tile_ex, valid, down_s, h, down_w):
    grid_spec = pltpu.PrefetchScalarGridSpec(
        num_scalar_prefetch=3,
        grid=(NDH, NT),
        in_specs=[
            pl.BlockSpec((M, FF), lambda dh, m, te, va, ds: (m, 0)),
            pl.BlockSpec((1, DH_BLK, FF),
                         lambda dh, m, te, va, ds: (te[m], dh, 0)),
        ],
        out_specs=pl.BlockSpec((M, DH_BLK), lambda dh, m, te, va, ds: (m, dh)),
    )
    return pl.pallas_call(
        _down_body,
        grid_spec=grid_spec,
        out_shape=jax.ShapeDtypeStruct((SPAD, D), jnp.float32),
    )(tile_ex, valid, down_s, h, down_w)


def _shared_body(x_ref, sg_ref, su_ref, sd_ref, out_ref):
    f = pl.program_id(1)

    @pl.when(f == 0)
    def _():
        out_ref[:] = jnp.zeros_like(out_ref)

    xb = x_ref[:]
    sg = sg_ref[:].astype(jnp.bfloat16)
    su = su_ref[:].astype(jnp.bfloat16)
    sd = sd_ref[:].astype(jnp.bfloat16)
    g = lax.dot_general(xb, sg, (((1,), (1,)), ((), ())),
                        preferred_element_type=jnp.float32)
    u = lax.dot_general(xb, su, (((1,), (1,)), ((), ())),
                        preferred_element_type=jnp.float32)
    h = ((g * (1.0 / (1.0 + jnp.exp(-g)))) * u).astype(jnp.bfloat16)
    out_ref[:] += lax.dot_general(h, sd, (((1,), (1,)), ((), ())),
                                  preferred_element_type=jnp.float32)


def _shared(x, shared_gate, shared_up, shared_down):
    return pl.pallas_call(
        _shared_body,
        grid=(T // TT7, FF // F7),
        in_specs=[
            pl.BlockSpec((TT7, D), lambda t, f: (t, 0)),
            pl.BlockSpec((F7, D), lambda t, f: (f, 0)),
            pl.BlockSpec((F7, D), lambda t, f: (f, 0)),
            pl.BlockSpec((D, F7), lambda t, f: (0, f)),
        ],
        out_specs=pl.BlockSpec((TT7, D), lambda t, f: (t, 0)),
        out_shape=jax.ShapeDtypeStruct((T, D), jnp.float32),
    )(x, shared_gate, shared_up, shared_down)


def _combine_body(sh_ref, y_ref, w_ref, out_ref):
    acc = sh_ref[:]
    for k in range(TOPK):
        acc = acc + w_ref[:, k:k + 1] * y_ref[:, k, :]
    out_ref[:] = acc


def _combine(shared_out, y3, w):
    return pl.pallas_call(
        _combine_body,
        grid=(T // TTC,),
        in_specs=[
            pl.BlockSpec((TTC, D), lambda i: (i, 0)),
            pl.BlockSpec((TTC, TOPK, D), lambda i: (i, 0, 0)),
            pl.BlockSpec((TTC, TOPK), lambda i: (i, 0)),
        ],
        out_specs=pl.BlockSpec((TTC, D), lambda i: (i, 0)),
        out_shape=jax.ShapeDtypeStruct((T, D), jnp.float32),
    )(shared_out, y3, w)


def kernel(x, router_weight, gate_w, up_w, down_w, gate_s, up_s, down_s,
           shared_gate, shared_up, shared_down):
    topk_idx, routing_w = _router(x, router_weight)
    pos, tile_ex, valid = _metadata(topk_idx)
    pos_flat = pos.reshape(S)
    tile_ex = tile_ex.reshape(NT)
    valid = valid.reshape(NT)

    xb = x.astype(jnp.bfloat16)
    xs = _sc_scatter(x, pos_flat)
    h = _gateup(tile_ex, valid, gate_s, up_s, xs, gate_w, up_w)
    ys = _down(tile_ex, valid, down_s, h, down_w)
    yg = _sc_gather(ys, pos_flat)

    sh = _shared(xb, shared_gate, shared_up, shared_down)
    return _combine(sh, yg.reshape(T, TOPK, D), routing_w)


# speedup vs baseline: 2.3554x; 1.0197x over previous
import functools

import jax
import jax.numpy as jnp
from jax import lax
from jax.experimental import pallas as pl
from jax.experimental.pallas import tpu as pltpu
from jax.experimental.pallas import tpu_sc as plsc

E = 64
TOPK = 8
D = 2048
FF = 5632
T = 4096

M = 256
S = T * TOPK
NT = (S + E * (M - 1) + M - 1) // M
SPAD = NT * M
F_BLK = 1408
NF = FF // F_BLK
DH_BLK = 1024
NDH = D // DH_BLK
TT1 = 512
C2 = 512
F7 = 256
TT7 = 1024
TTC = 256

NEG = -1e30

SC_NC = 2
SC_NS = 16
NW = SC_NC * SC_NS
PER_W = S // NW
B_SC = 16
N_IT = PER_W // B_SC


def _router_body(x_ref, rw_ref, idx_ref, w_ref):
    logits = lax.dot_general(x_ref[:], rw_ref[:], (((1,), (1,)), ((), ())),
                             preferred_element_type=jnp.float32)
    iota_e = lax.broadcasted_iota(jnp.int32, (TT1, E), 1)
    iota_k = lax.broadcasted_iota(jnp.int32, (TT1, TOPK), 1)
    vals = jnp.zeros((TT1, TOPK), jnp.float32)
    idxs = jnp.zeros((TT1, TOPK), jnp.int32)
    l = logits
    for k in range(TOPK):
        m = jnp.max(l, axis=1, keepdims=True)
        amax = jnp.min(jnp.where(l == m, iota_e, E), axis=1, keepdims=True)
        vals = jnp.where(iota_k == k, m, vals)
        idxs = jnp.where(iota_k == k, amax, idxs)
        l = jnp.where(iota_e == amax, NEG, l)
    ex = jnp.exp(vals - vals[:, 0:1])
    w_ref[:] = ex / jnp.sum(ex, axis=1, keepdims=True)
    idx_ref[:] = idxs


def _router(x, router_weight):
    return pl.pallas_call(
        _router_body,
        grid=(T // TT1,),
        in_specs=[
            pl.BlockSpec((TT1, D), lambda i: (i, 0)),
            pl.BlockSpec((E, D), lambda i: (0, 0)),
        ],
        out_specs=[
            pl.BlockSpec((TT1, TOPK), lambda i: (i, 0)),
            pl.BlockSpec((TT1, TOPK), lambda i: (i, 0)),
        ],
        out_shape=[
            jax.ShapeDtypeStruct((T, TOPK), jnp.int32),
            jax.ShapeDtypeStruct((T, TOPK), jnp.float32),
        ],
    )(x, router_weight)


def _meta_body(idx_ref, pos_ref, te_ref, valid_ref):
    idx = idx_ref[:]
    eqf = (idx[:, :, None] ==
           lax.broadcasted_iota(jnp.int32, (T, TOPK, E), 2)).astype(jnp.float32)
    a = jnp.sum(eqf, axis=1)
    ltri = (lax.broadcasted_iota(jnp.int32, (C2, C2), 0) >
            lax.broadcasted_iota(jnp.int32, (C2, C2), 1)).astype(jnp.float32)
    carry = jnp.zeros((1, E), jnp.float32)
    chunks = []
    for c in range(T // C2):
        ac = lax.slice(a, (c * C2, 0), ((c + 1) * C2, E))
        chunks.append(lax.dot_general(ltri, ac, (((1,), (0,)), ((), ())),
                                      preferred_element_type=jnp.float32)
                      + carry)
        carry = carry + jnp.sum(ac, axis=0, keepdims=True)
    rank = jnp.concatenate(chunks, axis=0)
    counts = carry
    counts_pad = jnp.floor((counts + (M - 1)) * (1.0 / M)) * M
    sut = (lax.broadcasted_iota(jnp.int32, (E, E), 0) <
           lax.broadcasted_iota(jnp.int32, (E, E), 1)).astype(jnp.float32)
    offs = lax.dot_general(counts_pad, sut, (((1,), (0,)), ((), ())),
                           preferred_element_type=jnp.float32)
    ends = offs + counts_pad
    total = jnp.sum(counts_pad, axis=1, keepdims=True)
    rank_sel = jnp.sum(eqf * rank[:, None, :], axis=2)
    offs_sel = jnp.sum(eqf * offs[:, None, :], axis=2)
    pos_ref[:] = (rank_sel + offs_sel).astype(jnp.int32)
    ident = (lax.broadcasted_iota(jnp.int32, (E, E), 0) ==
             lax.broadcasted_iota(jnp.int32, (E, E), 1)).astype(jnp.float32)
    ends_t = lax.dot_general(ident, ends, (((1,), (1,)), ((), ())),
                             preferred_element_type=jnp.float32)
    jstart = (lax.broadcasted_iota(jnp.int32, (1, NT), 1).astype(jnp.float32)
              * float(M))
    te_f = jnp.sum((ends_t <= jstart).astype(jnp.float32), axis=0,
                   keepdims=True)
    te_ref[:] = jnp.minimum(te_f, E - 1).astype(jnp.int32)
    valid_ref[:] = (jstart < total).astype(jnp.int32)


def _metadata(topk_idx):
    return pl.pallas_call(
        _meta_body,
        out_shape=[
            jax.ShapeDtypeStruct((T, TOPK), jnp.int32),
            jax.ShapeDtypeStruct((1, NT), jnp.int32),
            jax.ShapeDtypeStruct((1, NT), jnp.int32),
        ],
    )(topk_idx)


def _sc_scatter_body(x_hbm, pos_hbm, out_hbm, pos_v, buf, sem_g, sem_s):
    wid = lax.axis_index("s") * SC_NC + lax.axis_index("c")
    base = wid * PER_W
    pltpu.sync_copy(pos_hbm.at[pl.ds(base, PER_W)], pos_v)
    lane = lax.iota(jnp.int32, B_SC)

    def tok(i):
        return lax.shift_right_logical(base + i * B_SC + lane, 3)

    pltpu.async_copy(x_hbm.at[tok(0)], buf.at[0], sem_g)

    def body(i, _):
        slot = lax.rem(i, 2)
        nxt = jnp.minimum(i + 1, N_IT - 1)
        pltpu.make_async_copy(x_hbm.at[tok(i)], buf.at[slot], sem_g).wait()
        pltpu.async_copy(x_hbm.at[tok(nxt)], buf.at[1 - slot], sem_g)
        posv = pos_v[pl.ds(i * B_SC, B_SC)]
        pltpu.async_copy(buf.at[slot], out_hbm.at[posv], sem_s).wait()
        return 0

    lax.fori_loop(0, N_IT, body, 0)
    pltpu.make_async_copy(x_hbm.at[tok(N_IT - 1)],
                          buf.at[(N_IT - 1) % 2], sem_g).wait()


def _sc_scatter(x, pos_flat):
    return pl.kernel(
        _sc_scatter_body,
        out_type=jax.ShapeDtypeStruct((SPAD, D), jnp.float32),
        mesh=plsc.VectorSubcoreMesh(core_axis_name="c", subcore_axis_name="s"),
        scratch_types=[
            pltpu.VMEM((PER_W,), jnp.int32),
            pltpu.VMEM((2, B_SC, D), jnp.float32),
            pltpu.SemaphoreType.DMA,
            pltpu.SemaphoreType.DMA,
        ],
    )(x, pos_flat)


def _sc_gather_body(ys_hbm, pos_hbm, out_hbm, pos_v, buf, sem_g, sem_s):
    wid = lax.axis_index("s") * SC_NC + lax.axis_index("c")
    base = wid * PER_W
    pltpu.sync_copy(pos_hbm.at[pl.ds(base, PER_W)], pos_v)

    def posv(i):
        return pos_v[pl.ds(i * B_SC, B_SC)]

    pltpu.async_copy(ys_hbm.at[posv(0)], buf.at[0], sem_g)

    def body(i, _):
        slot = lax.rem(i, 2)
        nxt = jnp.minimum(i + 1, N_IT - 1)
        pltpu.make_async_copy(ys_hbm.at[posv(i)], buf.at[slot], sem_g).wait()
        pltpu.async_copy(ys_hbm.at[posv(nxt)], buf.at[1 - slot], sem_g)
        pltpu.async_copy(buf.at[slot], out_hbm.at[pl.ds(base + i * B_SC, B_SC)],
                         sem_s).wait()
        return 0

    lax.fori_loop(0, N_IT, body, 0)
    pltpu.make_async_copy(ys_hbm.at[posv(N_IT - 1)],
                          buf.at[N_IT % 2], sem_g).wait()


def _sc_gather(y_sorted, pos_flat):
    return pl.kernel(
        _sc_gather_body,
        out_type=jax.ShapeDtypeStruct((S, D), jnp.float32),
        mesh=plsc.VectorSubcoreMesh(core_axis_name="c", subcore_axis_name="s"),
        scratch_types=[
            pltpu.VMEM((PER_W,), jnp.int32),
            pltpu.VMEM((2, B_SC, D), jnp.float32),
            pltpu.SemaphoreType.DMA,
            pltpu.SemaphoreType.DMA,
        ],
    )(y_sorted, pos_flat)


def _gateup_body(te_sm, valid_sm, gs_sm, us_sm, xs_ref, gw_ref, uw_ref, h_ref):
    m = pl.program_id(1)

    @pl.when(valid_sm[m] == 1)
    def _():
        e = te_sm[m]
        xb = xs_ref[:].astype(jnp.bfloat16)
        gw = gw_ref[0].astype(jnp.bfloat16)
        uw = uw_ref[0].astype(jnp.bfloat16)
        g = lax.dot_general(xb, gw, (((1,), (1,)), ((), ())),
                            preferred_element_type=jnp.float32) * gs_sm[e]
        u = lax.dot_general(xb, uw, (((1,), (1,)), ((), ())),
                            preferred_element_type=jnp.float32) * us_sm[e]
        act = g * (1.0 / (1.0 + jnp.exp(-g)))
        h_ref[:] = (act * u).astype(jnp.bfloat16)


def _gateup(tile_ex, valid, gate_s, up_s, xs, gate_w, up_w):
    grid_spec = pltpu.PrefetchScalarGridSpec(
        num_scalar_prefetch=4,
        grid=(NF, NT),
        in_specs=[
            pl.BlockSpec((M, D), lambda f, m, te, va, gs, us: (m * va[m], 0)),
            pl.BlockSpec((1, F_BLK, D),
                         lambda f, m, te, va, gs, us: (te[m], f, 0)),
            pl.BlockSpec((1, F_BLK, D),
                         lambda f, m, te, va, gs, us: (te[m], f, 0)),
        ],
        out_specs=pl.BlockSpec((M, F_BLK), lambda f, m, te, va, gs, us: (m, f)),
    )
    return pl.pallas_call(
        _gateup_body,
        grid_spec=grid_spec,
        out_shape=jax.ShapeDtypeStruct((SPAD, FF), jnp.bfloat16),
    )(tile_ex, valid, gate_s, up_s, xs, gate_w, up_w)


def _down_body(te_sm, valid_sm, ds_sm, h_ref, dw_ref, y_ref):
    m = pl.program_id(1)

    @pl.when(valid_sm[m] == 1)
    def _():
        e = te_sm[m]
        h = h_ref[:]
        dw = dw_ref[0].astype(jnp.bfloat16)
        y_ref[:] = lax.dot_general(h, dw, (((1,), (1,)), ((), ())),
                                   preferred_element_type=jnp.float32) * ds_sm[e]


def _down(tile_ex, valid, down_s, h, down_w):
    grid_spec = pltpu.PrefetchScalarGridSpec(
        num_scalar_prefetch=3,
        grid=(NDH, NT),
        in_specs=[
            pl.BlockSpec((M, FF), lambda dh, m, te, va, ds: (m * va[m], 0)),
            pl.BlockSpec((1, DH_BLK, FF),
                         lambda dh, m, te, va, ds: (te[m], dh, 0)),
        ],
        out_specs=pl.BlockSpec((M, DH_BLK), lambda dh, m, te, va, ds: (m, dh)),
    )
    return pl.pallas_call(
        _down_body,
        grid_spec=grid_spec,
        out_shape=jax.ShapeDtypeStruct((SPAD, D), jnp.float32),
    )(tile_ex, valid, down_s, h, down_w)


def _shared_body(x_ref, sg_ref, su_ref, sd_ref, out_ref):
    f = pl.program_id(1)

    @pl.when(f == 0)
    def _():
        out_ref[:] = jnp.zeros_like(out_ref)

    xb = x_ref[:]
    sg = sg_ref[:].astype(jnp.bfloat16)
    su = su_ref[:].astype(jnp.bfloat16)
    sd = sd_ref[:].astype(jnp.bfloat16)
    g = lax.dot_general(xb, sg, (((1,), (1,)), ((), ())),
                        preferred_element_type=jnp.float32)
    u = lax.dot_general(xb, su, (((1,), (1,)), ((), ())),
                        preferred_element_type=jnp.float32)
    h = ((g * (1.0 / (1.0 + jnp.exp(-g)))) * u).astype(jnp.bfloat16)
    out_ref[:] += lax.dot_general(h, sd, (((1,), (1,)), ((), ())),
                                  preferred_element_type=jnp.float32)


def _shared(x, shared_gate, shared_up, shared_down):
    return pl.pallas_call(
        _shared_body,
        grid=(T // TT7, FF // F7),
        in_specs=[
            pl.BlockSpec((TT7, D), lambda t, f: (t, 0)),
            pl.BlockSpec((F7, D), lambda t, f: (f, 0)),
            pl.BlockSpec((F7, D), lambda t, f: (f, 0)),
            pl.BlockSpec((D, F7), lambda t, f: (0, f)),
        ],
        out_specs=pl.BlockSpec((TT7, D), lambda t, f: (t, 0)),
        out_shape=jax.ShapeDtypeStruct((T, D), jnp.float32),
    )(x, shared_gate, shared_up, shared_down)


def _combine_body(sh_ref, y_ref, w_ref, out_ref):
    acc = sh_ref[:]
    for k in range(TOPK):
        acc = acc + w_ref[:, k:k + 1] * y_ref[:, k, :]
    out_ref[:] = acc


def _combine(shared_out, y3, w):
    return pl.pallas_call(
        _combine_body,
        grid=(T // TTC,),
        in_specs=[
            pl.BlockSpec((TTC, D), lambda i: (i, 0)),
            pl.BlockSpec((TTC, TOPK, D), lambda i: (i, 0, 0)),
            pl.BlockSpec((TTC, TOPK), lambda i: (i, 0)),
        ],
        out_specs=pl.BlockSpec((TTC, D), lambda i: (i, 0)),
        out_shape=jax.ShapeDtypeStruct((T, D), jnp.float32),
    )(shared_out, y3, w)


def kernel(x, router_weight, gate_w, up_w, down_w, gate_s, up_s, down_s,
           shared_gate, shared_up, shared_down):
    topk_idx, routing_w = _router(x, router_weight)
    pos, tile_ex, valid = _metadata(topk_idx)
    pos_flat = pos.reshape(S)
    tile_ex = tile_ex.reshape(NT)
    valid = valid.reshape(NT)

    xb = x.astype(jnp.bfloat16)
    xs = _sc_scatter(x, pos_flat)
    h = _gateup(tile_ex, valid, gate_s, up_s, xs, gate_w, up_w)
    ys = _down(tile_ex, valid, down_s, h, down_w)
    yg = _sc_gather(ys, pos_flat)

    sh = _shared(xb, shared_gate, shared_up, shared_down)
    return _combine(sh, yg.reshape(T, TOPK, D), routing_w)
